# baseline (speedup 1.0000x reference)
import functools

import jax
import jax.numpy as jnp
from jax import lax
from jax.experimental import pallas as pl
from jax.experimental.pallas import tpu as pltpu
from jax.experimental.pallas import tpu_sc as plsc

BATCH = 16384
HIST = 50
D_MODEL = 64

NUM_CORES = 2
NUM_SUBCORES = 16
NW = NUM_CORES * NUM_SUBCORES
B_PER_W = BATCH // NW
UNIT = 256
N_UNITS = HIST * 2

_mesh = plsc.VectorSubcoreMesh(core_axis_name="c", subcore_axis_name="s")


@functools.partial(
    pl.kernel,
    out_type=jax.ShapeDtypeStruct((HIST, D_MODEL, BATCH), jnp.float32),
    mesh=_mesh,
    compiler_params=pltpu.CompilerParams(use_tc_tiling_on_sc=True,
                                         needs_layout_passes=False),
    scratch_types=[
        pltpu.VMEM((UNIT,), jnp.int32),
        pltpu.VMEM((UNIT,), jnp.int32),
        pltpu.VMEM((UNIT,), jnp.int32),
        pltpu.VMEM((UNIT,), jnp.int32),
        pltpu.VMEM((UNIT, 128), jnp.float32),
        pltpu.VMEM((UNIT, 128), jnp.float32),
        pltpu.VMEM((D_MODEL, 128), jnp.float32),
        pltpu.VMEM((D_MODEL, 128), jnp.float32),
        pltpu.VMEM((D_MODEL, 128), jnp.float32),
        pltpu.VMEM((D_MODEL, 128), jnp.float32),
        pltpu.SemaphoreType.DMA,
        pltpu.SemaphoreType.DMA,
        pltpu.SemaphoreType.DMA,
        pltpu.SemaphoreType.DMA,
    ],
)
def _embed_kernel(idx_hbm, t2_hbm, out_hbm,
                  ia0, ia1, p0, p1, r0, r1, s00, s01, s10, s11,
                  g0, g1, st0, st1):
    wid = lax.axis_index("s") * NUM_CORES + lax.axis_index("c")
    bstripe = wid * B_PER_W
    idx_v = (ia0, ia1)
    pair_v = (p0, p1)
    rows_v = (r0, r1)
    slab_v = ((s00, s01), (s10, s11))
    gsem = (g0, g1)
    ssem = (st0, st1)

    def flat_base(u):
        h = u >> 1
        half = u & 1
        return h * BATCH + bstripe + half * UNIT

    def load_idx(u, b):
        pltpu.sync_copy(idx_hbm.at[pl.ds(flat_base(u), UNIT)], idx_v[b])
        for i in range(UNIT // 16):
            pair_v[b][pl.ds(16 * i, 16)] = (
                idx_v[b][pl.ds(16 * i, 16)] >> 1)

    def start_gather(b):
        pltpu.make_async_copy(t2_hbm.at[pair_v[b]], rows_v[b],
                              gsem[b]).start()

    def wait_gather(b):
        pltpu.make_async_copy(t2_hbm.at[pair_v[b]], rows_v[b],
                              gsem[b]).wait()

    def consume(u, b):
        h = u >> 1
        half = u & 1
        for jb in range(2):
            slab = slab_v[b][jb]
            hoff = []
            for j in range(8):
                vi = idx_v[b][pl.ds(jb * 128 + 16 * j, 16)]
                hoff.append((vi & 1) * 64)

            def body(d, carry):
                for j in range(8):
                    ridx = (lax.broadcasted_iota(jnp.int32, (16,), 0)
                            + (jb * 128 + 16 * j))
                    v = plsc.load_gather(rows_v[b], [ridx, hoff[j] + d])
                    slab[d, pl.ds(16 * j, 16)] = v
                return carry

            lax.fori_loop(0, D_MODEL, body, 0)
            bcol = bstripe + half * UNIT + jb * 128
            pltpu.make_async_copy(
                slab, out_hbm.at[h, :, pl.ds(bcol, 128)], ssem[b]).start()

    def wait_stores(b):
        for jb in range(2):
            pltpu.make_async_copy(
                slab_v[b][jb], out_hbm.at[0, :, pl.ds(bstripe, 128)],
                ssem[b]).wait()

    load_idx(0, 0)
    start_gather(0)
    load_idx(1, 1)
    start_gather(1)
    wait_gather(0)
    consume(0, 0)
    load_idx(2, 0)
    start_gather(0)
    wait_gather(1)
    consume(1, 1)
    load_idx(3, 1)
    start_gather(1)

    def loop(g, carry):
        for b in range(2):
            u = 2 * g + 2 + b
            wait_gather(b)
            wait_stores(b)
            consume(u, b)
            load_idx(u + 2, b)
            start_gather(b)
        return carry

    lax.fori_loop(0, (N_UNITS - 4) // 2, loop, 0)

    for b in range(2):
        u = N_UNITS - 2 + b
        wait_gather(b)
        wait_stores(b)
        consume(u, b)
    wait_stores(0)
    wait_stores(1)


def kernel(indices, table):
    idx_t = indices.T.reshape(-1)
    t2 = table.reshape(500000, 128)
    out_p = _embed_kernel(idx_t, t2)
    return out_p.transpose(2, 0, 1)


# speedup vs baseline: 1.6811x; 1.6811x over previous
import functools

import jax
import jax.numpy as jnp
from jax import lax
from jax.experimental import pallas as pl
from jax.experimental.pallas import tpu as pltpu
from jax.experimental.pallas import tpu_sc as plsc

BATCH = 16384
HIST = 50
D_MODEL = 64
B_TOTAL = BATCH * HIST

NUM_CORES = 2
NUM_SUBCORES = 16
NW = NUM_CORES * NUM_SUBCORES
B_PER_W = B_TOTAL // NW
CHUNK = 512
N_CHUNKS = B_PER_W // CHUNK

_mesh = plsc.VectorSubcoreMesh(core_axis_name="c", subcore_axis_name="s")


@functools.partial(
    pl.kernel,
    out_type=jax.ShapeDtypeStruct((B_TOTAL, D_MODEL), jnp.float32),
    mesh=_mesh,
    compiler_params=pltpu.CompilerParams(use_tc_tiling_on_sc=False),
    scratch_types=[
        pltpu.VMEM((B_PER_W,), jnp.int32),
        pltpu.VMEM((B_PER_W,), jnp.int32),
        pltpu.VMEM((2, CHUNK, D_MODEL), jnp.float32),
        pltpu.SemaphoreType.DMA,
        pltpu.SemaphoreType.DMA,
        pltpu.SemaphoreType.DMA,
        pltpu.SemaphoreType.DMA,
    ],
)
def _sc_gather(idx_hbm, sidx_hbm, table_hbm, out_hbm,
               idx_v, sidx_v, rows_v, g0, g1, s0, s1):
    wid = lax.axis_index("s") * NUM_CORES + lax.axis_index("c")
    base = wid * B_PER_W
    gsem = (g0, g1)
    ssem = (s0, s1)

    def start_gather(i, b):
        pltpu.make_async_copy(
            table_hbm.at[idx_v.at[pl.ds(i * CHUNK, CHUNK)]],
            rows_v.at[b], gsem[b]).start()

    def wait_gather(b):
        pltpu.make_async_copy(
            table_hbm.at[idx_v.at[pl.ds(0, CHUNK)]],
            rows_v.at[b], gsem[b]).wait()

    def start_store(i, b):
        pltpu.make_async_copy(
            rows_v.at[b], out_hbm.at[sidx_v.at[pl.ds(i * CHUNK, CHUNK)]],
            ssem[b]).start()

    def wait_store(b):
        pltpu.make_async_copy(
            rows_v.at[b], out_hbm.at[sidx_v.at[pl.ds(0, CHUNK)]],
            ssem[b]).wait()

    pltpu.sync_copy(idx_hbm.at[pl.ds(base, B_PER_W)], idx_v)
    pltpu.sync_copy(sidx_hbm.at[pl.ds(base, B_PER_W)], sidx_v)

    start_gather(0, 0)
    wait_gather(0)
    start_store(0, 0)
    start_gather(1, 1)

    def body(g, carry):
        i1 = 2 * g + 1
        wait_gather(1)
        start_store(i1, 1)
        wait_store(0)
        start_gather(i1 + 1, 0)
        i2 = 2 * g + 2
        wait_gather(0)
        start_store(i2, 0)
        wait_store(1)
        start_gather(i2 + 1, 1)
        return carry

    lax.fori_loop(0, (N_CHUNKS - 2) // 2, body, 0)

    wait_gather(1)
    start_store(N_CHUNKS - 1, 1)
    wait_store(0)
    wait_store(1)


BB = 1024


def _tc_body(x_ref, o_ref):
    x = x_ref[0]
    o_ref[0] = x[:, :D_MODEL].T
    o_ref[1] = x[:, D_MODEL:].T


def _tc_transpose(o5):
    return pl.pallas_call(
        _tc_body,
        out_shape=jax.ShapeDtypeStruct((HIST, D_MODEL, BATCH), jnp.float32),
        grid=(HIST // 2, BATCH // BB),
        in_specs=[pl.BlockSpec((1, BB, 128), lambda j, i: (j, i, 0))],
        out_specs=pl.BlockSpec((2, D_MODEL, BB), lambda j, i: (j, 0, i)),
    )(o5)


def kernel(indices, table):
    flat_idx = indices.reshape(-1)
    i = jnp.arange(B_TOTAL, dtype=jnp.int32)
    b = i // HIST
    h = i - b * HIST
    sidx = ((h >> 1) * BATCH + b) * 2 + (h & 1)
    o3 = _sc_gather(flat_idx, sidx, table)
    o5 = o3.reshape(HIST // 2, BATCH, 128)
    out_p = _tc_transpose(o5)
    return out_p.transpose(2, 0, 1)


# speedup vs baseline: 1.9154x; 1.1394x over previous
import functools

import jax
import jax.numpy as jnp
from jax import lax
from jax.experimental import pallas as pl
from jax.experimental.pallas import tpu as pltpu
from jax.experimental.pallas import tpu_sc as plsc

BATCH = 16384
HIST = 50
D_MODEL = 64
B_TOTAL = BATCH * HIST

NUM_CORES = 2
NUM_SUBCORES = 16
NW = NUM_CORES * NUM_SUBCORES
B_PER_W = B_TOTAL // NW
CHUNK = 512
N_CHUNKS = B_PER_W // CHUNK

_mesh = plsc.VectorSubcoreMesh(core_axis_name="c", subcore_axis_name="s")


@functools.partial(
    pl.kernel,
    out_type=jax.ShapeDtypeStruct((B_TOTAL, D_MODEL), jnp.float32),
    mesh=_mesh,
    compiler_params=pltpu.CompilerParams(use_tc_tiling_on_sc=False),
    scratch_types=[
        pltpu.VMEM((B_PER_W,), jnp.int32),
        pltpu.VMEM((B_PER_W,), jnp.int32),
        pltpu.VMEM((2, CHUNK, D_MODEL), jnp.float32),
        pltpu.SemaphoreType.DMA,
        pltpu.SemaphoreType.DMA,
        pltpu.SemaphoreType.DMA,
        pltpu.SemaphoreType.DMA,
    ],
)
def _sc_gather(idx_hbm, sidx_hbm, table_hbm, out_hbm,
               idx_v, sidx_v, rows_v, g0, g1, s0, s1):
    wid = lax.axis_index("s") * NUM_CORES + lax.axis_index("c")
    base = wid * B_PER_W
    gsem = (g0, g1)
    ssem = (s0, s1)

    def start_gather(i, b):
        pltpu.make_async_copy(
            table_hbm.at[idx_v.at[pl.ds(i * CHUNK, CHUNK)]],
            rows_v.at[b], gsem[b]).start()

    def wait_gather(b):
        pltpu.make_async_copy(
            table_hbm.at[idx_v.at[pl.ds(0, CHUNK)]],
            rows_v.at[b], gsem[b]).wait()

    def start_store(i, b):
        pltpu.make_async_copy(
            rows_v.at[b], out_hbm.at[sidx_v.at[pl.ds(i * CHUNK, CHUNK)]],
            ssem[b]).start()

    def wait_store(b):
        pltpu.make_async_copy(
            rows_v.at[b], out_hbm.at[sidx_v.at[pl.ds(0, CHUNK)]],
            ssem[b]).wait()

    pltpu.sync_copy(idx_hbm.at[pl.ds(base, B_PER_W)], idx_v)
    pltpu.sync_copy(sidx_hbm.at[pl.ds(base, B_PER_W)], sidx_v)

    start_gather(0, 0)
    wait_gather(0)
    start_store(0, 0)
    start_gather(1, 1)

    def body(g, carry):
        i1 = 2 * g + 1
        wait_gather(1)
        start_store(i1, 1)
        wait_store(0)
        start_gather(i1 + 1, 0)
        i2 = 2 * g + 2
        wait_gather(0)
        start_store(i2, 0)
        wait_store(1)
        start_gather(i2 + 1, 1)
        return carry

    lax.fori_loop(0, (N_CHUNKS - 2) // 2, body, 0)

    wait_gather(1)
    start_store(N_CHUNKS - 1, 1)
    wait_store(0)
    wait_store(1)


BB = 2048


def _tc_body(x_ref, o_ref):
    x = x_ref[0]
    xt = x.T
    o_ref[0] = xt[:D_MODEL]
    o_ref[1] = xt[D_MODEL:]


def _tc_transpose(o5):
    return pl.pallas_call(
        _tc_body,
        out_shape=jax.ShapeDtypeStruct((HIST, D_MODEL, BATCH), jnp.float32),
        grid=(HIST // 2, BATCH // BB),
        in_specs=[pl.BlockSpec((1, BB, 128), lambda j, i: (j, i, 0))],
        out_specs=pl.BlockSpec((2, D_MODEL, BB), lambda j, i: (j, 0, i)),
    )(o5)


def kernel(indices, table):
    flat_idx = indices.reshape(-1)
    i = jnp.arange(B_TOTAL, dtype=jnp.int32)
    b = i // HIST
    h = i - b * HIST
    sidx = ((h >> 1) * BATCH + b) * 2 + (h & 1)
    o3 = _sc_gather(flat_idx, sidx, table)
    o5 = o3.reshape(HIST // 2, BATCH, 128)
    out_p = _tc_transpose(o5)
    return out_p.transpose(2, 0, 1)


# speedup vs baseline: 2.0454x; 1.0679x over previous
import functools

import jax
import jax.numpy as jnp
from jax import lax
from jax.experimental import pallas as pl
from jax.experimental.pallas import tpu as pltpu
from jax.experimental.pallas import tpu_sc as plsc

BATCH = 16384
HIST = 50
D_MODEL = 64
B_TOTAL = BATCH * HIST

NUM_CORES = 2
NUM_SUBCORES = 16
NW = NUM_CORES * NUM_SUBCORES
B_PER_W = B_TOTAL // NW
CHUNK = 512
N_CHUNKS = B_PER_W // CHUNK

_mesh = plsc.VectorSubcoreMesh(core_axis_name="c", subcore_axis_name="s")


@functools.partial(
    pl.kernel,
    out_type=jax.ShapeDtypeStruct((B_TOTAL, D_MODEL), jnp.float32),
    mesh=_mesh,
    compiler_params=pltpu.CompilerParams(use_tc_tiling_on_sc=False),
    scratch_types=[
        pltpu.VMEM((B_PER_W,), jnp.int32),
        pltpu.VMEM((B_PER_W,), jnp.int32),
        pltpu.VMEM((2, CHUNK, D_MODEL), jnp.float32),
        pltpu.SemaphoreType.DMA,
        pltpu.SemaphoreType.DMA,
        pltpu.SemaphoreType.DMA,
        pltpu.SemaphoreType.DMA,
    ],
)
def _sc_gather(idx_hbm, sidx_hbm, table_hbm, out_hbm,
               idx_v, sidx_v, rows_v, g0, g1, s0, s1):
    wid = lax.axis_index("s") * NUM_CORES + lax.axis_index("c")
    base = wid * B_PER_W
    gsem = (g0, g1)
    ssem = (s0, s1)

    def start_gather(i, b):
        pltpu.make_async_copy(
            table_hbm.at[idx_v.at[pl.ds(i * CHUNK, CHUNK)]],
            rows_v.at[b], gsem[b]).start()

    def wait_gather(b):
        pltpu.make_async_copy(
            table_hbm.at[idx_v.at[pl.ds(0, CHUNK)]],
            rows_v.at[b], gsem[b]).wait()

    def start_store(i, b):
        pltpu.make_async_copy(
            rows_v.at[b], out_hbm.at[sidx_v.at[pl.ds(i * CHUNK, CHUNK)]],
            ssem[b]).start()

    def wait_store(b):
        pltpu.make_async_copy(
            rows_v.at[b], out_hbm.at[sidx_v.at[pl.ds(0, CHUNK)]],
            ssem[b]).wait()

    pltpu.sync_copy(idx_hbm.at[pl.ds(base, B_PER_W)], idx_v)
    pltpu.sync_copy(sidx_hbm.at[pl.ds(base, B_PER_W)], sidx_v)

    start_gather(0, 0)
    wait_gather(0)
    start_store(0, 0)
    start_gather(1, 1)

    def body(g, carry):
        i1 = 2 * g + 1
        wait_gather(1)
        start_store(i1, 1)
        wait_store(0)
        start_gather(i1 + 1, 0)
        i2 = 2 * g + 2
        wait_gather(0)
        start_store(i2, 0)
        wait_store(1)
        start_gather(i2 + 1, 1)
        return carry

    lax.fori_loop(0, (N_CHUNKS - 2) // 2, body, 0)

    wait_gather(1)
    start_store(N_CHUNKS - 1, 1)
    wait_store(0)
    wait_store(1)


BB = 4096


def _tc_body(x_ref, o_ref):
    o_ref[0] = x_ref[0].T


def _tc_transpose(o5):
    out = pl.pallas_call(
        _tc_body,
        out_shape=jax.ShapeDtypeStruct((HIST // 2, 128, BATCH), jnp.float32),
        grid=(HIST // 2, BATCH // BB),
        in_specs=[pl.BlockSpec((1, BB, 128), lambda j, i: (j, i, 0))],
        out_specs=pl.BlockSpec((1, 128, BB), lambda j, i: (j, 0, i)),
    )(o5)
    return out.reshape(HIST, D_MODEL, BATCH)


def kernel(indices, table):
    flat_idx = indices.reshape(-1)
    i = jnp.arange(B_TOTAL, dtype=jnp.int32)
    b = i // HIST
    h = i - b * HIST
    sidx = ((h >> 1) * BATCH + b) * 2 + (h & 1)
    o3 = _sc_gather(flat_idx, sidx, table)
    o5 = o3.reshape(HIST // 2, BATCH, 128)
    out_p = _tc_transpose(o5)
    return out_p.transpose(2, 0, 1)
